# baseline (speedup 1.0000x reference)
import functools

import jax
import jax.numpy as jnp
from jax import lax
from jax.experimental import pallas as pl
from jax.experimental.pallas import tpu as pltpu
from jax.experimental.pallas import tpu_sc as plsc

NUM_CODEBOOKS = 32
HIDDEN = 2048
LANES = 16
NQ = 4
QROWS = NUM_CODEBOOKS // NQ


def _sc_embed_sum(ids_flat, table, offsets, *, num_positions):
    mesh = plsc.VectorSubcoreMesh(core_axis_name="c", subcore_axis_name="s")
    num_cores = mesh.num_cores
    n_workers = mesh.num_cores * mesh.num_subcores
    pos_per_worker = num_positions // n_workers
    idx_per_worker = pos_per_worker * NUM_CODEBOOKS

    @functools.partial(
        pl.kernel,
        out_type=jax.ShapeDtypeStruct((num_positions, HIDDEN), jnp.float32),
        mesh=mesh,
        scratch_types=[
            pltpu.VMEM((idx_per_worker,), jnp.int32),
            pltpu.VMEM((NUM_CODEBOOKS,), jnp.int32),
            pltpu.VMEM((NQ, QROWS, HIDDEN), jnp.float32),
            pltpu.VMEM((2, HIDDEN), jnp.float32),
            pltpu.SemaphoreType.DMA,
            pltpu.SemaphoreType.DMA,
            pltpu.SemaphoreType.DMA,
            pltpu.SemaphoreType.DMA,
            pltpu.SemaphoreType.DMA,
            pltpu.SemaphoreType.DMA,
        ],
    )
    def k(
        ids_hbm, table_hbm, offs_hbm, out_hbm,
        idx_v, offs_v, bufs, acc_v,
        sem0, sem1, sem2, sem3, osem0, osem1,
    ):
        sems = (sem0, sem1, sem2, sem3)
        wid = lax.axis_index("s") * num_cores + lax.axis_index("c")
        base_idx = wid * idx_per_worker
        base_pos = wid * pos_per_worker

        pltpu.sync_copy(ids_hbm.at[pl.ds(base_idx, idx_per_worker)], idx_v)
        pltpu.sync_copy(offs_hbm, offs_v)
        off_lo = offs_v[pl.ds(0, LANES)]
        off_hi = offs_v[pl.ds(LANES, LANES)]

        def add_offsets(p, carry):
            o = pl.multiple_of(p * NUM_CODEBOOKS, 8)
            idx_v[pl.ds(o, LANES)] += off_lo
            idx_v[pl.ds(o + LANES, LANES)] += off_hi
            return carry

        lax.fori_loop(0, pos_per_worker, add_offsets, 0, unroll=4)

        def gather_quarter(pos, q):
            o = pl.multiple_of(pos * NUM_CODEBOOKS + q * QROWS, 8)
            pltpu.async_copy(
                table_hbm.at[idx_v.at[pl.ds(o, QROWS)]], bufs.at[q], sems[q]
            )

        def drain_quarter(q):
            pltpu.make_async_copy(
                table_hbm.at[idx_v.at[pl.ds(0, QROWS)]], bufs.at[q], sems[q]
            ).wait()

        def reduce_quarter(q, slot, first):
            def rh(h, carry2):
                ho = pl.multiple_of(h * LANES, 8)
                a = bufs[q, 0, pl.ds(ho, LANES)]
                for r in range(1, QROWS):
                    a = a + bufs[q, r, pl.ds(ho, LANES)]
                if first:
                    acc_v[slot, pl.ds(ho, LANES)] = a
                else:
                    plsc.addupdate(acc_v.at[slot, pl.ds(ho, LANES)], a)
                return carry2

            lax.fori_loop(0, HIDDEN // LANES, rh, 0, unroll=2)

        def drain_out(slot, osem):
            pltpu.make_async_copy(
                acc_v.at[slot], out_hbm.at[base_pos], osem
            ).wait()

        for q in range(NQ):
            gather_quarter(0, q)

        def body(g, carry):
            for p, slot, osem in ((2 * g, 0, osem0), (2 * g + 1, 1, osem1)):
                pn = jnp.minimum(p + 1, pos_per_worker - 1)
                @pl.when(g > 0)
                def _():
                    drain_out(slot, osem)

                for q in range(NQ):
                    drain_quarter(q)
                    reduce_quarter(q, slot, first=(q == 0))
                    gather_quarter(pn, q)
                pltpu.async_copy(acc_v.at[slot], out_hbm.at[base_pos + p], osem)
            return carry

        lax.fori_loop(0, pos_per_worker // 2, body, 0)

        for q in range(NQ):
            drain_quarter(q)
        drain_out(0, osem0)
        drain_out(1, osem1)

    return k(ids_flat, table, offsets)


def kernel(input_ids, embed_table, audio_tokens_offsets):
    b, s, c = input_ids.shape
    ids_flat = input_ids.reshape(-1).astype(jnp.int32)
    offs = audio_tokens_offsets.astype(jnp.int32)
    out = _sc_embed_sum(ids_flat, embed_table, offs, num_positions=b * s)
    return out.reshape(b, s, embed_table.shape[1])


# speedup vs baseline: 1.5258x; 1.5258x over previous
import functools

import jax
import jax.numpy as jnp
from jax import lax
from jax.experimental import pallas as pl
from jax.experimental.pallas import tpu as pltpu
from jax.experimental.pallas import tpu_sc as plsc

NUM_CODEBOOKS = 32
HIDDEN = 2048
LANES = 16
HALF = NUM_CODEBOOKS // 2


def _sc_embed_sum(ids_flat, table, offsets, *, num_positions):
    mesh = plsc.VectorSubcoreMesh(core_axis_name="c", subcore_axis_name="s")
    num_cores = mesh.num_cores
    n_workers = mesh.num_cores * mesh.num_subcores
    pos_per_worker = num_positions // n_workers
    idx_per_worker = pos_per_worker * NUM_CODEBOOKS

    @functools.partial(
        pl.kernel,
        out_type=jax.ShapeDtypeStruct((num_positions, HIDDEN), jnp.float32),
        mesh=mesh,
        scratch_types=[
            pltpu.VMEM((idx_per_worker,), jnp.int32),
            pltpu.VMEM((NUM_CODEBOOKS,), jnp.int32),
            pltpu.VMEM((HALF, HIDDEN), jnp.float32),
            pltpu.VMEM((HALF, HIDDEN), jnp.float32),
            pltpu.VMEM((2, HIDDEN), jnp.float32),
            pltpu.SemaphoreType.DMA,
            pltpu.SemaphoreType.DMA,
            pltpu.SemaphoreType.DMA,
            pltpu.SemaphoreType.DMA,
        ],
    )
    def k(
        ids_hbm, table_hbm, offs_hbm, out_hbm,
        idx_v, offs_v, buf_a, buf_b, acc_v,
        sem_a, sem_b, osem0, osem1,
    ):
        wid = lax.axis_index("s") * num_cores + lax.axis_index("c")
        base_idx = wid * idx_per_worker
        base_pos = wid * pos_per_worker

        pltpu.sync_copy(ids_hbm.at[pl.ds(base_idx, idx_per_worker)], idx_v)
        pltpu.sync_copy(offs_hbm, offs_v)
        off_lo = offs_v[pl.ds(0, LANES)]
        off_hi = offs_v[pl.ds(LANES, LANES)]

        def add_offsets(p, carry):
            o = pl.multiple_of(p * NUM_CODEBOOKS, 8)
            idx_v[pl.ds(o, LANES)] += off_lo
            idx_v[pl.ds(o + LANES, LANES)] += off_hi
            return carry

        lax.fori_loop(0, pos_per_worker, add_offsets, 0, unroll=4)

        def gather_half(flat_off, buf, sem):
            return pltpu.async_copy(
                table_hbm.at[idx_v.at[pl.ds(flat_off, HALF)]], buf, sem
            )

        def drain_half(buf, sem):
            pltpu.make_async_copy(
                table_hbm.at[idx_v.at[pl.ds(0, HALF)]], buf, sem
            ).wait()

        GRP = 8

        def reduce_into(buf, slot, first):
            def rh(hg, carry2):
                ho = pl.multiple_of(hg * (GRP * LANES), 8)
                accs = [
                    buf[0, pl.ds(ho + j * LANES, LANES)] for j in range(GRP)
                ]
                for r in range(1, HALF):
                    for j in range(GRP):
                        accs[j] = accs[j] + buf[r, pl.ds(ho + j * LANES, LANES)]
                for j in range(GRP):
                    if first:
                        acc_v[slot, pl.ds(ho + j * LANES, LANES)] = accs[j]
                    else:
                        plsc.addupdate(
                            acc_v.at[slot, pl.ds(ho + j * LANES, LANES)], accs[j]
                        )
                return carry2

            lax.fori_loop(0, HIDDEN // (GRP * LANES), rh, 0)

        def drain_out(slot, osem):
            pltpu.make_async_copy(
                acc_v.at[slot], out_hbm.at[base_pos], osem
            ).wait()

        gather_half(0, buf_a, sem_a)

        def body(g, carry):
            for p, slot, osem in ((2 * g, 0, osem0), (2 * g + 1, 1, osem1)):
                o = pl.multiple_of(p * NUM_CODEBOOKS, 8)
                drain_half(buf_a, sem_a)
                gather_half(o + HALF, buf_b, sem_b)
                @pl.when(g > 0)
                def _():
                    drain_out(slot, osem)

                reduce_into(buf_a, slot, first=True)
                pn = jnp.minimum(p + 1, pos_per_worker - 1)
                drain_half(buf_b, sem_b)
                gather_half(pn * NUM_CODEBOOKS, buf_a, sem_a)
                reduce_into(buf_b, slot, first=False)
                pltpu.async_copy(acc_v.at[slot], out_hbm.at[base_pos + p], osem)
            return carry

        lax.fori_loop(0, pos_per_worker // 2, body, 0)

        drain_half(buf_a, sem_a)
        drain_out(0, osem0)
        drain_out(1, osem1)

    return k(ids_flat, table, offsets)


def kernel(input_ids, embed_table, audio_tokens_offsets):
    b, s, c = input_ids.shape
    ids_flat = input_ids.reshape(-1).astype(jnp.int32)
    offs = audio_tokens_offsets.astype(jnp.int32)
    out = _sc_embed_sum(ids_flat, embed_table, offs, num_positions=b * s)
    return out.reshape(b, s, embed_table.shape[1])


# speedup vs baseline: 2.1924x; 1.4368x over previous
import functools

import jax
import jax.numpy as jnp
from jax import lax
from jax.experimental import pallas as pl
from jax.experimental.pallas import tpu as pltpu
from jax.experimental.pallas import tpu_sc as plsc

NUM_CODEBOOKS = 32
HIDDEN = 2048
LANES = 16
NQ = 4
QROWS = NUM_CODEBOOKS // NQ
GRP = 8


def _sc_embed_sum(ids_flat, table, offsets, *, num_positions):
    mesh = plsc.VectorSubcoreMesh(core_axis_name="c", subcore_axis_name="s")
    num_cores = mesh.num_cores
    n_workers = mesh.num_cores * mesh.num_subcores
    pos_per_worker = num_positions // n_workers
    idx_per_worker = pos_per_worker * NUM_CODEBOOKS

    @functools.partial(
        pl.kernel,
        out_type=jax.ShapeDtypeStruct((num_positions, HIDDEN), jnp.float32),
        mesh=mesh,
        scratch_types=[
            pltpu.VMEM((idx_per_worker,), jnp.int32),
            pltpu.VMEM((NUM_CODEBOOKS,), jnp.int32),
            pltpu.VMEM((NQ, QROWS, HIDDEN), jnp.float32),
            pltpu.VMEM((2, HIDDEN), jnp.float32),
            pltpu.SemaphoreType.DMA,
            pltpu.SemaphoreType.DMA,
            pltpu.SemaphoreType.DMA,
            pltpu.SemaphoreType.DMA,
            pltpu.SemaphoreType.DMA,
            pltpu.SemaphoreType.DMA,
        ],
    )
    def k(
        ids_hbm, table_hbm, offs_hbm, out_hbm,
        idx_v, offs_v, bufs, acc_v,
        sem0, sem1, sem2, sem3, osem0, osem1,
    ):
        sems = (sem0, sem1, sem2, sem3)
        wid = lax.axis_index("s") * num_cores + lax.axis_index("c")
        base_idx = wid * idx_per_worker
        base_pos = wid * pos_per_worker

        pltpu.sync_copy(ids_hbm.at[pl.ds(base_idx, idx_per_worker)], idx_v)
        pltpu.sync_copy(offs_hbm, offs_v)
        off_lo = offs_v[pl.ds(0, LANES)]
        off_hi = offs_v[pl.ds(LANES, LANES)]

        def add_offsets(p, carry):
            o = pl.multiple_of(p * NUM_CODEBOOKS, 8)
            idx_v[pl.ds(o, LANES)] += off_lo
            idx_v[pl.ds(o + LANES, LANES)] += off_hi
            return carry

        lax.fori_loop(0, pos_per_worker, add_offsets, 0, unroll=4)

        def gather_quarter(pos, q):
            o = pl.multiple_of(pos * NUM_CODEBOOKS + q * QROWS, 8)
            pltpu.async_copy(
                table_hbm.at[idx_v.at[pl.ds(o, QROWS)]], bufs.at[q], sems[q]
            )

        def drain_quarter(q):
            pltpu.make_async_copy(
                table_hbm.at[idx_v.at[pl.ds(0, QROWS)]], bufs.at[q], sems[q]
            ).wait()

        def reduce_quarter(q, slot, first):
            def rh(hg, carry2):
                ho = pl.multiple_of(hg * (GRP * LANES), 8)
                accs = [
                    bufs[q, 0, pl.ds(ho + j * LANES, LANES)] for j in range(GRP)
                ]
                for r in range(1, QROWS):
                    for j in range(GRP):
                        accs[j] = accs[j] + bufs[
                            q, r, pl.ds(ho + j * LANES, LANES)
                        ]
                for j in range(GRP):
                    if first:
                        acc_v[slot, pl.ds(ho + j * LANES, LANES)] = accs[j]
                    else:
                        plsc.addupdate(
                            acc_v.at[slot, pl.ds(ho + j * LANES, LANES)],
                            accs[j],
                        )
                return carry2

            lax.fori_loop(0, HIDDEN // (GRP * LANES), rh, 0)

        def drain_out(slot, osem):
            pltpu.make_async_copy(
                acc_v.at[slot], out_hbm.at[base_pos], osem
            ).wait()

        for q in range(NQ):
            gather_quarter(0, q)

        def body(g, carry):
            for p, slot, osem in ((2 * g, 0, osem0), (2 * g + 1, 1, osem1)):
                pn = jnp.minimum(p + 1, pos_per_worker - 1)
                @pl.when(g > 0)
                def _():
                    drain_out(slot, osem)

                for q in range(NQ):
                    drain_quarter(q)
                    reduce_quarter(q, slot, first=(q == 0))
                    gather_quarter(pn, q)
                pltpu.async_copy(acc_v.at[slot], out_hbm.at[base_pos + p], osem)
            return carry

        lax.fori_loop(0, pos_per_worker // 2, body, 0)

        for q in range(NQ):
            drain_quarter(q)
        drain_out(0, osem0)
        drain_out(1, osem1)

    return k(ids_flat, table, offsets)


def kernel(input_ids, embed_table, audio_tokens_offsets):
    b, s, c = input_ids.shape
    ids_flat = input_ids.reshape(-1).astype(jnp.int32)
    offs = audio_tokens_offsets.astype(jnp.int32)
    out = _sc_embed_sum(ids_flat, embed_table, offs, num_positions=b * s)
    return out.reshape(b, s, embed_table.shape[1])
